# Initial kernel scaffold; baseline (speedup 1.0000x reference)
import jax
import jax.numpy as jnp
from jax.experimental import pallas as pl


def kernel(vision_features, W):
    raise NotImplementedError("write your pallas kernel here")


# speedup vs baseline: 1.2263x; 1.2263x over previous
import jax
import jax.numpy as jnp
from jax.experimental import pallas as pl

_N_PATCHES = 16
_BLK = 512


def _body(vf_ref, w_ref, out_ref):
    blk, d = vf_ref.shape
    ppb = w_ref.shape[0]
    rpp = blk // ppb
    x = vf_ref[...].reshape(ppb, rpp, d) + w_ref[...]
    out_ref[...] = x.reshape(blk, d)


def kernel(vision_features, W):
    B, S, D = vision_features.shape
    rpp = S // _N_PATCHES
    ppb = _BLK // rpp
    R = B * S
    vf = vision_features.reshape(R, D)
    w3 = W.reshape(W.shape[0], 1, D)
    nblk = R // _BLK
    wblocks = _N_PATCHES // ppb
    out = pl.pallas_call(
        _body,
        grid=(nblk,),
        in_specs=[
            pl.BlockSpec((_BLK, D), lambda k: (k, 0)),
            pl.BlockSpec((ppb, 1, D), lambda k: (k % wblocks, 0, 0)),
        ],
        out_specs=pl.BlockSpec((_BLK, D), lambda k: (k, 0)),
        out_shape=jax.ShapeDtypeStruct((R, D), vision_features.dtype),
    )(vf, w3)
    return out.reshape(B, S, D)


# speedup vs baseline: 1.2399x; 1.0111x over previous
import jax
import jax.numpy as jnp
from jax.experimental import pallas as pl

_N_PATCHES = 16
_BLK = 1024


def _body(vf_ref, w_ref, out_ref):
    blk, d = vf_ref.shape
    ppb = w_ref.shape[0]
    rpp = blk // ppb
    x = vf_ref[...].reshape(ppb, rpp, d) + w_ref[...]
    out_ref[...] = x.reshape(blk, d)


def kernel(vision_features, W):
    B, S, D = vision_features.shape
    rpp = S // _N_PATCHES
    ppb = _BLK // rpp
    R = B * S
    vf = vision_features.reshape(R, D)
    w3 = W.reshape(W.shape[0], 1, D)
    nblk = R // _BLK
    wblocks = _N_PATCHES // ppb
    out = pl.pallas_call(
        _body,
        grid=(nblk,),
        in_specs=[
            pl.BlockSpec((_BLK, D), lambda k: (k, 0)),
            pl.BlockSpec((ppb, 1, D), lambda k: (k % wblocks, 0, 0)),
        ],
        out_specs=pl.BlockSpec((_BLK, D), lambda k: (k, 0)),
        out_shape=jax.ShapeDtypeStruct((R, D), vision_features.dtype),
    )(vf, w3)
    return out.reshape(B, S, D)
